# baseline (speedup 1.0000x reference)
import functools

import jax
import jax.numpy as jnp
from jax import lax
from jax.experimental import pallas as pl
from jax.experimental.pallas import tpu as pltpu
from jax.experimental.pallas import tpu_sc as plsc

_N = 10000
_E = 160000
_IN = 256
_HID = 512
_OUT = 256
_D = _OUT
_HOPS = 10
_ALPHA = 0.1

_NC = 2
_NS = 16
_L = 16
_NW = _NC * _NS
_RPT = 320
_NPAD = _NW * _RPT
_CAPE = 8192
_CHUNK = 32
_EPAD = _E + _CAPE


def _mlp_body(x_ref, w1_ref, b1_ref, w2_ref, b2_ref, o_ref):
    h = jnp.dot(x_ref[...], w1_ref[...], preferred_element_type=jnp.float32)
    h = jnp.maximum(h + b1_ref[...], 0.0)
    o = jnp.dot(h, w2_ref[...], preferred_element_type=jnp.float32)
    o_ref[...] = o + b2_ref[...]


def _mlp(X, W1, b1, W2, b2):
    BR = 1000
    return pl.pallas_call(
        _mlp_body,
        grid=(_N // BR,),
        in_specs=[
            pl.BlockSpec((BR, _IN), lambda i: (i, 0)),
            pl.BlockSpec((_IN, _HID), lambda i: (0, 0)),
            pl.BlockSpec((1, _HID), lambda i: (0, 0)),
            pl.BlockSpec((_HID, _OUT), lambda i: (0, 0)),
            pl.BlockSpec((1, _OUT), lambda i: (0, 0)),
        ],
        out_specs=pl.BlockSpec((BR, _OUT), lambda i: (i, 0)),
        out_shape=jax.ShapeDtypeStruct((_N, _OUT), jnp.float32),
    )(X, W1, b1.reshape(1, _HID), W2, b2.reshape(1, _OUT))


def _make_hop():
    mesh = plsc.VectorSubcoreMesh(
        core_axis_name="c", subcore_axis_name="s",
        num_cores=_NC, num_subcores=_NS)

    @functools.partial(
        pl.kernel,
        out_type=jax.ShapeDtypeStruct((_NPAD, _D), jnp.float32),
        mesh=mesh,
        scratch_types=[
            pltpu.VMEM((_RPT, _D), jnp.float32),
            pltpu.VMEM((_CAPE,), jnp.int32),
            pltpu.VMEM((_CAPE,), jnp.int32),
            pltpu.VMEM((_CAPE,), jnp.float32),
            pltpu.VMEM((_CHUNK, _D), jnp.float32),
            pltpu.VMEM((272,), jnp.int32),
            pltpu.SemaphoreType.DMA,
        ],
        compiler_params=pltpu.CompilerParams(
            use_tc_tiling_on_sc=False, needs_layout_passes=False),
    )
    def hop(z_hbm, z0_hbm, col_hbm, val_hbm, dst_hbm, st_hbm, out_hbm,
            acc, colseg, dstseg, valseg, rows, st_v, sem):
        wid = lax.axis_index("s") * _NC + lax.axis_index("c")
        base = wid * _RPT
        iota = lax.iota(jnp.int32, _L)

        pltpu.sync_copy(st_hbm, st_v)
        start = st_v[pl.ds(pl.multiple_of(wid * 8, 8), _L)][0]
        end = st_v[pl.ds(pl.multiple_of(wid * 8 + 8, 8), _L)][0]

        zero = jnp.zeros((_L,), jnp.float32)

        @pl.loop(0, _RPT)
        def _zero_row(r):
            row_ref = acc.at[r]
            for g in range(_D // _L):
                row_ref[pl.ds(g * _L, _L)] = zero

        astart = start & (-8)
        total = end - astart
        nseg = lax.div(total + (_CAPE - 1), _CAPE)

        @pl.loop(0, nseg)
        def _seg(si):
            soff = pl.multiple_of(astart + si * _CAPE, 8)
            pltpu.sync_copy(col_hbm.at[pl.ds(soff, _CAPE)], colseg)
            pltpu.sync_copy(dst_hbm.at[pl.ds(soff, _CAPE)], dstseg)
            pltpu.sync_copy(val_hbm.at[pl.ds(soff, _CAPE)], valseg)
            seg_n = jnp.minimum(end - soff, _CAPE)
            nch = lax.div(seg_n + (_CHUNK - 1), _CHUNK)

            @pl.loop(0, nch)
            def _chunk(ci):
                ebase = ci * _CHUNK
                pltpu.async_copy(
                    z_hbm.at[colseg.at[pl.ds(ebase, _CHUNK)]], rows, sem
                ).wait()
                goff = soff + ebase
                for g in range(_CHUNK // _L):
                    eidx = (goff + g * _L) + iota
                    valid = (eidx >= start) & (eidx < end)
                    dstv = dstseg[pl.ds(ebase + g * _L, _L)]
                    valv = valseg[pl.ds(ebase + g * _L, _L)]
                    a = jnp.where(valid, valv * (1.0 - _ALPHA), 0.0)
                    ldst = jnp.where(valid, dstv - base, 0)
                    rowv = iota + (g * _L)

                    def _col(cc, colv):
                        r = plsc.load_gather(rows, [rowv, colv])
                        plsc.addupdate_scatter(acc, [ldst, colv], a * r)
                        return colv + 1

                    plsc.parallel_loop(
                        0, _D, 1, unroll=8,
                        carry=jnp.zeros((_L,), jnp.int32),
                    )(_col)

        @pl.loop(0, _RPT // _CHUNK)
        def _wb(kk):
            pltpu.sync_copy(z0_hbm.at[pl.ds(base + kk * _CHUNK, _CHUNK)], rows)

            @pl.loop(0, _CHUNK)
            def _comb(r):
                arow = acc.at[kk * _CHUNK + r]
                zrow = rows.at[r]
                for g in range(_D // _L):
                    sl = pl.ds(g * _L, _L)
                    arow[sl] = arow[sl] + _ALPHA * zrow[sl]

        pltpu.sync_copy(acc, out_hbm.at[pl.ds(base, _RPT)])

    return hop


_hop = _make_hop()


def kernel(X, edge_index, A_val, W1, b1, W2, b2):
    Z0 = _mlp(X, W1, b1, W2, b2)

    row = edge_index[0]
    col = edge_index[1]
    order = jnp.argsort(row)
    sdst = row[order]
    scol = col[order]
    sval = A_val[order]
    starts = jnp.searchsorted(
        sdst, jnp.arange(33, dtype=jnp.int32) * _RPT, side="left"
    ).astype(jnp.int32)
    starts = jnp.concatenate([starts, jnp.zeros((1,), jnp.int32)])
    starts = jnp.concatenate(
        [starts.reshape(34, 1), jnp.zeros((34, 7), jnp.int32)], axis=1
    ).reshape(272)
    scol = jnp.concatenate([scol, jnp.zeros((_EPAD - _E,), jnp.int32)])
    sdst = jnp.concatenate([sdst, jnp.zeros((_EPAD - _E,), jnp.int32)])
    sval = jnp.concatenate([sval, jnp.zeros((_EPAD - _E,), jnp.float32)])

    z0p = jnp.pad(Z0, ((0, _NPAD - _N), (0, 0)))
    z = z0p
    for _ in range(_HOPS):
        z = _hop(z, z0p, scol, sval, sdst, starts)
    return z[:_N]


# speedup vs baseline: 2.9076x; 2.9076x over previous
import functools

import jax
import jax.numpy as jnp
from jax import lax
from jax.experimental import pallas as pl
from jax.experimental.pallas import tpu as pltpu
from jax.experimental.pallas import tpu_sc as plsc

_N = 10000
_E = 160000
_IN = 256
_HID = 512
_OUT = 256
_D = _OUT
_HOPS = 10
_ALPHA = 0.1

_NC = 2
_NS = 16
_L = 16
_NW = _NC * _NS
_RPT = 320
_NPAD = _NW * _RPT
_CAPE = 8192
_CHUNK = 32
_EPAD = _E + _CAPE


def _mlp_body(x_ref, w1_ref, b1_ref, w2_ref, b2_ref, o_ref, oa_ref):
    h = jnp.dot(x_ref[...], w1_ref[...], preferred_element_type=jnp.float32)
    h = jnp.maximum(h + b1_ref[...], 0.0)
    o = jnp.dot(h, w2_ref[...], preferred_element_type=jnp.float32)
    o = o + b2_ref[...]
    o_ref[...] = o
    oa_ref[...] = o * _ALPHA


def _mlp(X, W1, b1, W2, b2):
    BR = 1000
    return pl.pallas_call(
        _mlp_body,
        grid=(_N // BR,),
        in_specs=[
            pl.BlockSpec((BR, _IN), lambda i: (i, 0)),
            pl.BlockSpec((_IN, _HID), lambda i: (0, 0)),
            pl.BlockSpec((1, _HID), lambda i: (0, 0)),
            pl.BlockSpec((_HID, _OUT), lambda i: (0, 0)),
            pl.BlockSpec((1, _OUT), lambda i: (0, 0)),
        ],
        out_specs=[
            pl.BlockSpec((BR, _OUT), lambda i: (i, 0)),
            pl.BlockSpec((BR, _OUT), lambda i: (i, 0)),
        ],
        out_shape=[
            jax.ShapeDtypeStruct((_N, _OUT), jnp.float32),
            jax.ShapeDtypeStruct((_N, _OUT), jnp.float32),
        ],
    )(X, W1, b1.reshape(1, _HID), W2, b2.reshape(1, _OUT))


def _make_hop():
    mesh = plsc.VectorSubcoreMesh(
        core_axis_name="c", subcore_axis_name="s",
        num_cores=_NC, num_subcores=_NS)

    @functools.partial(
        pl.kernel,
        out_type=jax.ShapeDtypeStruct((_NPAD, _D), jnp.float32),
        mesh=mesh,
        scratch_types=[
            pltpu.VMEM((_RPT, _D), jnp.float32),
            pltpu.VMEM((_CAPE,), jnp.int32),
            pltpu.VMEM((_CAPE,), jnp.int32),
            pltpu.VMEM((_CAPE,), jnp.float32),
            pltpu.VMEM((_CHUNK, _D), jnp.float32),
            pltpu.VMEM((_CHUNK, _D), jnp.float32),
            pltpu.VMEM((272,), jnp.int32),
            pltpu.SemaphoreType.DMA,
            pltpu.SemaphoreType.DMA,
        ],
        compiler_params=pltpu.CompilerParams(
            use_tc_tiling_on_sc=False, needs_layout_passes=False),
    )
    def hop(z_hbm, z0a_hbm, col_hbm, val_hbm, dst_hbm, st_hbm, out_hbm,
            acc, colseg, dstseg, valseg, rows0, rows1, st_v, sem0, sem1):
        wid = lax.axis_index("s") * _NC + lax.axis_index("c")
        base = wid * _RPT
        iota = lax.iota(jnp.int32, _L)

        pltpu.sync_copy(st_hbm, st_v)
        start = st_v[pl.ds(pl.multiple_of(wid * 8, 8), _L)][0]
        end = st_v[pl.ds(pl.multiple_of(wid * 8 + 8, 8), _L)][0]

        pltpu.sync_copy(z0a_hbm.at[pl.ds(base, _RPT)], acc)

        astart = start & (-8)
        total = end - astart
        nseg = lax.div(total + (_CAPE - 1), _CAPE)

        def _gather(cs, ci, rbuf, sem):
            src = z_hbm.at[colseg.at[pl.ds(ci * _CHUNK, _CHUNK)]]
            pltpu.async_copy(src, rbuf, sem)

        def _gwait(cs, ci, rbuf, sem):
            src = z_hbm.at[colseg.at[pl.ds(ci * _CHUNK, _CHUNK)]]
            pltpu.make_async_copy(src, rbuf, sem).wait()

        def _process(cs, ci, rbuf):
            ebase = ci * _CHUNK
            for g in range(_CHUNK // _L):
                off = ebase + g * _L
                eidx = (cs + off) + iota
                valid = (eidx >= start) & (eidx < end)
                dstv = dstseg[pl.ds(off, _L)]
                valv = valseg[pl.ds(off, _L)]
                a = jnp.where(valid, valv, 0.0)
                ldst = jnp.clip(dstv - base, 0, _RPT - 1)
                for j in range(_L):
                    av = jnp.broadcast_to(a[j], (_L,))
                    rowref = rbuf.at[g * _L + j]
                    accrow = acc.at[ldst[j]]
                    for k in range(_D // _L):
                        sl = pl.ds(k * _L, _L)
                        plsc.addupdate(accrow.at[sl], av * rowref[sl])

        @pl.loop(0, nseg)
        def _seg(si):
            soff = pl.multiple_of(astart + si * _CAPE, 8)
            pltpu.sync_copy(col_hbm.at[pl.ds(soff, _CAPE)], colseg)
            pltpu.sync_copy(dst_hbm.at[pl.ds(soff, _CAPE)], dstseg)
            pltpu.sync_copy(val_hbm.at[pl.ds(soff, _CAPE)], valseg)
            seg_n = jnp.minimum(end - soff, _CAPE)
            nch = lax.div(seg_n + (_CHUNK - 1), _CHUNK)
            nch2 = lax.div(nch + 1, 2) * 2

            _gather(soff, 0, rows0, sem0)

            @pl.loop(0, nch2, step=2)
            def _c(ci):
                _gather(soff, ci + 1, rows1, sem1)
                _gwait(soff, ci, rows0, sem0)
                _process(soff, ci, rows0)

                @pl.when(ci + 2 < nch2)
                def _():
                    _gather(soff, ci + 2, rows0, sem0)

                _gwait(soff, ci + 1, rows1, sem1)
                _process(soff, ci + 1, rows1)

        pltpu.sync_copy(acc, out_hbm.at[pl.ds(base, _RPT)])

    return hop


_hop = _make_hop()


def kernel(X, edge_index, A_val, W1, b1, W2, b2):
    Z0, Z0a = _mlp(X, W1, b1, W2, b2)

    row = edge_index[0]
    col = edge_index[1]
    order = jnp.argsort(row)
    sdst = row[order]
    scol = col[order]
    sval = A_val[order] * (1.0 - _ALPHA)
    starts = jnp.searchsorted(
        sdst, jnp.arange(33, dtype=jnp.int32) * _RPT, side="left"
    ).astype(jnp.int32)
    starts = jnp.concatenate([starts, jnp.zeros((1,), jnp.int32)])
    starts = jnp.concatenate(
        [starts.reshape(34, 1), jnp.zeros((34, 7), jnp.int32)], axis=1
    ).reshape(272)
    scol = jnp.concatenate([scol, jnp.zeros((_EPAD - _E,), jnp.int32)])
    sdst = jnp.concatenate([sdst, jnp.zeros((_EPAD - _E,), jnp.int32)])
    sval = jnp.concatenate([sval, jnp.zeros((_EPAD - _E,), jnp.float32)])

    z0p = jnp.pad(Z0, ((0, _NPAD - _N), (0, 0)))
    z0ap = jnp.pad(Z0a, ((0, _NPAD - _N), (0, 0)))
    z = z0p
    for _ in range(_HOPS):
        z = _hop(z, z0ap, scol, sval, sdst, starts)
    return z[:_N]
